# baseline (speedup 1.0000x reference)
import jax
import jax.numpy as jnp
from jax.experimental import pallas as pl
from jax.experimental.pallas import tpu as pltpu

_K = 8
_H = 16
_T = 2048
_S = 2048
_SAMPLED_HEAD = 12
_SLAB = 8
_NCHUNK = 8
_HC = _H // _NCHUNK


def _update_kernel(full_ref, slab_ref, out_ref, newx_ref, csems, wsem):
    for i in range(_NCHUNK):
        pltpu.make_async_copy(
            full_ref.at[pl.ds(i * _HC, _HC)],
            out_ref.at[pl.ds(i * _HC, _HC)],
            csems.at[i],
        ).start()

    x = slab_ref[:, _SLAB - 1, :]

    m = jnp.max(x, axis=-1, keepdims=True)
    idx = jax.lax.broadcasted_iota(jnp.int32, x.shape, 1)
    arg = jnp.min(jnp.where(x == m, idx, _S), axis=-1)

    eq = (arg[:, None] == arg[None, :]).astype(jnp.int32)
    maxcount = jnp.max(jnp.sum(eq, axis=1))
    cond = maxcount <= _K

    row = x[_SAMPLED_HEAD, :]
    newx_ref[:, 0, :] = jnp.where(
        cond, jnp.broadcast_to(row[None, :], x.shape), x
    )

    for i in range(_NCHUNK):
        pltpu.make_async_copy(
            full_ref.at[pl.ds(i * _HC, _HC)],
            out_ref.at[pl.ds(i * _HC, _HC)],
            csems.at[i],
        ).wait()
    slice_cp = pltpu.make_async_copy(
        newx_ref, out_ref.at[:, pl.ds(_T - 1, 1), :], wsem
    )
    slice_cp.start()
    slice_cp.wait()


def kernel(attention_weight):
    aw = attention_weight.reshape(_H, _T, _S)
    last_blk = (_T - _SLAB) // _SLAB
    out = pl.pallas_call(
        _update_kernel,
        grid=(1,),
        in_specs=[
            pl.BlockSpec(memory_space=pltpu.MemorySpace.HBM),
            pl.BlockSpec((_H, _SLAB, _S), lambda i: (0, last_blk, 0)),
        ],
        out_specs=pl.BlockSpec(memory_space=pltpu.MemorySpace.HBM),
        out_shape=jax.ShapeDtypeStruct((_H, _T, _S), jnp.float32),
        scratch_shapes=[
            pltpu.VMEM((_H, 1, _S), jnp.float32),
            pltpu.SemaphoreType.DMA((_NCHUNK,)),
            pltpu.SemaphoreType.DMA,
        ],
    )(aw, aw)
    return out.reshape(1, _H, _T, _S)


# speedup vs baseline: 48.9650x; 48.9650x over previous
import jax
import jax.numpy as jnp
from jax.experimental import pallas as pl
from jax.experimental.pallas import tpu as pltpu

_K = 8
_H = 16
_T = 2048
_S = 2048
_SAMPLED_HEAD = 12
_SLAB = 8
_BR = 1024
_NB = (_H * _T) // _BR


def _copy_patch_kernel(flat_ref, slab_ref, out_ref, newx_ref):
    i = pl.program_id(0)

    @pl.when(i == 0)
    def _():
        x = slab_ref[:, _SLAB - 1, :]
        m = jnp.max(x, axis=-1, keepdims=True)
        idx = jax.lax.broadcasted_iota(jnp.int32, x.shape, 1)
        arg = jnp.min(jnp.where(x == m, idx, _S), axis=-1)
        eq = (arg[:, None] == arg[None, :]).astype(jnp.int32)
        maxcount = jnp.max(jnp.sum(eq, axis=1))
        cond = maxcount <= _K
        row = x[_SAMPLED_HEAD, :]
        newx_ref[...] = jnp.where(
            cond, jnp.broadcast_to(row[None, :], x.shape), x
        )

    out_ref[...] = flat_ref[...]

    @pl.when((i % 2) == 1)
    def _():
        h = (i - 1) // 2
        out_ref[pl.ds(_BR - 1, 1), :] = newx_ref[pl.ds(h, 1), :]


def kernel(attention_weight):
    flat = attention_weight.reshape(_H * _T, _S)
    slab3 = attention_weight.reshape(_H, _T, _S)
    last_blk = (_T - _SLAB) // _SLAB
    out = pl.pallas_call(
        _copy_patch_kernel,
        grid=(_NB,),
        in_specs=[
            pl.BlockSpec((_BR, _S), lambda i: (i, 0)),
            pl.BlockSpec((_H, _SLAB, _S), lambda i: (0, last_blk, 0)),
        ],
        out_specs=pl.BlockSpec((_BR, _S), lambda i: (i, 0)),
        out_shape=jax.ShapeDtypeStruct((_H * _T, _S), jnp.float32),
        scratch_shapes=[pltpu.VMEM((_H, _S), jnp.float32)],
    )(flat, slab3)
    return out.reshape(1, _H, _T, _S)
